# baseline (speedup 1.0000x reference)
import jax
import jax.numpy as jnp
from jax.experimental import pallas as pl
from jax.experimental.pallas import tpu as pltpu
from jax.experimental.pallas import tpu_sc as plsc

_N = 2048
_TILE = 256
_NTILES = _N // _TILE
_W = 128
_NSTRIPS = _N // _W
_ALPHA = 1000.0
_BIGKEY = 3.0e38
_EPS = 1e-6
_NTASK = 16
_MASK = 2047


_BPG = 4


def _tc_body(x1_ref, x2t_ref, e1_ref, i1_ref, e2_ref, i2_ref):
    lanec = [jax.lax.broadcasted_iota(jnp.int32, (_TILE, _W), 1) + k * _W
             for k in range(_NSTRIPS)]
    rowg = [jax.lax.broadcasted_iota(jnp.int32, (_TILE, _W), 0) + t * _TILE
            for t in range(_NTILES)]

    for bb in range(_BPG):
        ckrun = [jnp.full((_TILE // 16, _W), _BIGKEY, dtype=jnp.float32)
                 for _ in range(_NSTRIPS)]
        for t in range(_NTILES):
            r0 = t * _TILE
            ax = x1_ref[bb, pl.ds(r0, _TILE), 0:1]
            ay = x1_ref[bb, pl.ds(r0, _TILE), 1:2]
            az = x1_ref[bb, pl.ds(r0, _TILE), 2:3]
            rkey = jnp.full((_TILE, _W), _BIGKEY, dtype=jnp.float32)
            for k in range(_NSTRIPS):
                c0 = k * _W
                bx = x2t_ref[bb, 0:1, pl.ds(c0, _W)]
                by = x2t_ref[bb, 1:2, pl.ds(c0, _W)]
                bz = x2t_ref[bb, 2:3, pl.ds(c0, _W)]
                dx = ax - bx
                dy = ay - by
                dz = az - bz
                d = dx * dx + dy * dy + dz * dz
                dq = jax.lax.bitcast_convert_type(d, jnp.int32) & ~_MASK
                kr = jax.lax.bitcast_convert_type(dq | lanec[k], jnp.float32)
                rkey = jnp.minimum(rkey, kr)
                kc = jax.lax.bitcast_convert_type(dq | rowg[t], jnp.float32)
                ckrun[k] = jnp.minimum(
                    ckrun[k], jnp.min(kc.reshape(16, _TILE // 16, _W), axis=0))
            rk = jnp.min(rkey, axis=1, keepdims=True)
            rkb = jax.lax.bitcast_convert_type(rk, jnp.int32)
            i1_ref[bb, pl.ds(r0, _TILE), :] = rkb & _MASK
            rdq = jax.lax.bitcast_convert_type(rkb & ~_MASK, jnp.float32)
            e1_ref[bb, pl.ds(r0, _TILE), :] = jnp.exp(-rdq * _ALPHA)

        ck = jnp.concatenate(
            [jnp.min(c, axis=0, keepdims=True) for c in ckrun], axis=1)
        ckb = jax.lax.bitcast_convert_type(ck, jnp.int32)
        i2_ref[bb, :, :] = ckb & _MASK
        cdq = jax.lax.bitcast_convert_type(ckb & ~_MASK, jnp.float32)
        e2_ref[bb, :, :] = jnp.exp(-cdq * _ALPHA)


def _sc_density(i1, e1, i2, e2):
    mesh = plsc.VectorSubcoreMesh(core_axis_name="c", subcore_axis_name="s")

    @pl.kernel(
        out_type=jax.ShapeDtypeStruct((_NTASK, 16), jnp.float32),
        mesh=mesh,
        scratch_types=[
            pltpu.VMEM((_N,), jnp.int32),
            pltpu.VMEM((_N,), jnp.float32),
            pltpu.VMEM((_N,), jnp.int32),
            pltpu.VMEM((_N,), jnp.float32),
            pltpu.VMEM((_N,), jnp.float32),
            pltpu.VMEM((_N,), jnp.float32),
            pltpu.VMEM((16,), jnp.float32),
            pltpu.VMEM_SHARED((8 * _N,), jnp.float32),
            pltpu.VMEM_SHARED((8 * _N,), jnp.float32),
        ],
    )
    def k(i1_hbm, e1_hbm, i2_hbm, e2_hbm, o_hbm, idx_v, e_v, off_v, ones_v,
          c_v, s_v, acc_v, csh, ssh):
        cid = jax.lax.axis_index("c")
        sid = jax.lax.axis_index("s")

        @pl.when(sid < 8)
        def _():
            task = cid * 8 + sid
            base = sid * _N

            @pl.when(task < 8)
            def _():
                pltpu.sync_copy(i1_hbm.at[task], idx_v)
                pltpu.sync_copy(e1_hbm.at[task], e_v)

            @pl.when(task >= 8)
            def _():
                pltpu.sync_copy(i2_hbm.at[task - 8], idx_v)
                pltpu.sync_copy(e2_hbm.at[task - 8], e_v)

            zeros16 = jnp.zeros((16,), jnp.float32)
            ones16 = jnp.ones((16,), jnp.float32)

            @pl.loop(0, _N, step=16)
            def _(i):
                ones_v[pl.ds(i, 16)] = zeros16

            pltpu.sync_copy(ones_v, csh.at[pl.ds(base, _N)])
            pltpu.sync_copy(ones_v, ssh.at[pl.ds(base, _N)])

            @pl.loop(0, _N, step=16)
            def _(i):
                ones_v[pl.ds(i, 16)] = ones16
                off_v[pl.ds(i, 16)] = idx_v[pl.ds(i, 16)] + base

            pltpu.sync_copy(ones_v, csh.at[off_v], add=True)
            pltpu.sync_copy(e_v, ssh.at[off_v], add=True)
            pltpu.sync_copy(csh.at[pl.ds(base, _N)], c_v)
            pltpu.sync_copy(ssh.at[pl.ds(base, _N)], s_v)
            acc_v[...] = zeros16

            @pl.loop(0, _N, step=16)
            def _(i):
                acc_v[...] = acc_v[...] + (
                    s_v[pl.ds(i, 16)] / (c_v[pl.ds(i, 16)] + _EPS))

            pltpu.sync_copy(acc_v, o_hbm.at[task])

    return k(i1, e1, i2, e2)


def kernel(xyz1, xyz2):
    B = xyz1.shape[0]
    x2t = jnp.transpose(xyz2, (0, 2, 1))
    e1, i1, e2, i2 = pl.pallas_call(
        _tc_body,
        grid=(B // _BPG,),
        in_specs=[
            pl.BlockSpec((_BPG, _N, 3), lambda b: (b, 0, 0)),
            pl.BlockSpec((_BPG, 3, _N), lambda b: (b, 0, 0)),
        ],
        out_specs=[
            pl.BlockSpec((_BPG, _N, 1), lambda b: (b, 0, 0)),
            pl.BlockSpec((_BPG, _N, 1), lambda b: (b, 0, 0)),
            pl.BlockSpec((_BPG, 1, _N), lambda b: (b, 0, 0)),
            pl.BlockSpec((_BPG, 1, _N), lambda b: (b, 0, 0)),
        ],
        out_shape=[
            jax.ShapeDtypeStruct((B, _N, 1), jnp.float32),
            jax.ShapeDtypeStruct((B, _N, 1), jnp.int32),
            jax.ShapeDtypeStruct((B, 1, _N), jnp.float32),
            jax.ShapeDtypeStruct((B, 1, _N), jnp.int32),
        ],
    )(xyz1, x2t)
    parts = _sc_density(i1.reshape(B, _N), e1.reshape(B, _N),
                        i2.reshape(B, _N), e2.reshape(B, _N))
    return 1.0 - jnp.sum(parts) / (2.0 * _N * B)


# speedup vs baseline: 1.1441x; 1.1441x over previous
import dataclasses

import jax
import jax.numpy as jnp
from jax.experimental import pallas as pl
from jax.experimental.pallas import tpu as pltpu
from jax.experimental.pallas import tpu_sc as plsc

_N = 2048
_TILE = 256
_NTILES = _N // _TILE
_W = 128
_NSTRIPS = _N // _W
_ALPHA = 1000.0
_BIGKEY = 3.0e38
_EPS = 1e-6
_NTASK = 16
_MASK = 2047


def _tc_body(x1_ref, x2t_ref, k1_ref, k2_ref):
    lanec = [jax.lax.broadcasted_iota(jnp.int32, (_TILE, _W), 1) + k * _W
             for k in range(_NSTRIPS)]
    rowg = [jax.lax.broadcasted_iota(jnp.int32, (_TILE, _W), 0) + t * _TILE
            for t in range(_NTILES)]
    ckrun = [jnp.full((_TILE // 16, _W), _BIGKEY, dtype=jnp.float32)
             for _ in range(_NSTRIPS)]

    for t in range(_NTILES):
        r0 = t * _TILE
        ax = x1_ref[pl.ds(r0, _TILE), 0:1]
        ay = x1_ref[pl.ds(r0, _TILE), 1:2]
        az = x1_ref[pl.ds(r0, _TILE), 2:3]
        rkey = jnp.full((_TILE, _W), _BIGKEY, dtype=jnp.float32)
        for k in range(_NSTRIPS):
            c0 = k * _W
            bx = x2t_ref[0:1, pl.ds(c0, _W)]
            by = x2t_ref[1:2, pl.ds(c0, _W)]
            bz = x2t_ref[2:3, pl.ds(c0, _W)]
            dx = ax - bx
            dy = ay - by
            dz = az - bz
            d = dx * dx + dy * dy + dz * dz
            dq = jax.lax.bitcast_convert_type(d, jnp.int32) & ~_MASK
            kr = jax.lax.bitcast_convert_type(dq | lanec[k], jnp.float32)
            rkey = jnp.minimum(rkey, kr)
            kc = jax.lax.bitcast_convert_type(dq | rowg[t], jnp.float32)
            ckrun[k] = jnp.minimum(
                ckrun[k], jnp.min(kc.reshape(16, _TILE // 16, _W), axis=0))
        k1_ref[pl.ds(r0, _TILE), :] = jnp.min(rkey, axis=1, keepdims=True)

    k2_ref[:, :] = jnp.concatenate(
        [jnp.min(c, axis=0, keepdims=True) for c in ckrun], axis=1)


def _sc_density(k1, k2):
    mesh = plsc.VectorSubcoreMesh(core_axis_name="c", subcore_axis_name="s")
    cp = pltpu.CompilerParams()
    if "needs_layout_passes" in pltpu.CompilerParams.__dataclass_fields__:
        cp = dataclasses.replace(cp, needs_layout_passes=False)

    @pl.kernel(
        out_type=jax.ShapeDtypeStruct((_NTASK, 16), jnp.float32),
        mesh=mesh,
        compiler_params=cp,
        scratch_types=[
            pltpu.VMEM((_N,), jnp.float32),
            pltpu.VMEM((_N,), jnp.float32),
            pltpu.VMEM((_N,), jnp.int32),
            pltpu.VMEM((_N,), jnp.float32),
            pltpu.VMEM((_N,), jnp.float32),
            pltpu.VMEM((_N,), jnp.float32),
            pltpu.VMEM((16,), jnp.float32),
            pltpu.VMEM_SHARED((8 * _N,), jnp.float32),
            pltpu.VMEM_SHARED((8 * _N,), jnp.float32),
        ],
    )
    def k(k1_hbm, k2_hbm, o_hbm, key_v, e_v, off_v, ones_v,
          c_v, s_v, acc_v, csh, ssh):
        cid = jax.lax.axis_index("c")
        sid = jax.lax.axis_index("s")

        @pl.when(sid < 8)
        def _():
            task = cid * 8 + sid
            base = sid * _N

            @pl.when(task < 8)
            def _():
                pltpu.sync_copy(k1_hbm.at[task], key_v)

            @pl.when(task >= 8)
            def _():
                pltpu.sync_copy(k2_hbm.at[task - 8], key_v)

            zeros16 = jnp.zeros((16,), jnp.float32)
            ones16 = jnp.ones((16,), jnp.float32)

            @pl.loop(0, _N, step=16)
            def _(i):
                ones_v[pl.ds(i, 16)] = zeros16

            pltpu.sync_copy(ones_v, csh.at[pl.ds(base, _N)])
            pltpu.sync_copy(ones_v, ssh.at[pl.ds(base, _N)])

            @pl.loop(0, _N, step=16)
            def _(i):
                ones_v[pl.ds(i, 16)] = ones16
                kb = plsc.bitcast(key_v[pl.ds(i, 16)], jnp.int32)
                off_v[pl.ds(i, 16)] = (kb & _MASK) + base
                dq = plsc.bitcast(kb & ~_MASK, jnp.float32)
                e_v[pl.ds(i, 16)] = jnp.exp(-dq * _ALPHA)

            pltpu.sync_copy(ones_v, csh.at[off_v], add=True)
            pltpu.sync_copy(e_v, ssh.at[off_v], add=True)
            pltpu.sync_copy(csh.at[pl.ds(base, _N)], c_v)
            pltpu.sync_copy(ssh.at[pl.ds(base, _N)], s_v)
            acc_v[...] = zeros16

            @pl.loop(0, _N, step=16)
            def _(i):
                acc_v[...] = acc_v[...] + (
                    s_v[pl.ds(i, 16)] / (c_v[pl.ds(i, 16)] + _EPS))

            pltpu.sync_copy(acc_v, o_hbm.at[task])

    return k(k1, k2)


def kernel(xyz1, xyz2):
    B = xyz1.shape[0]
    x2t = jnp.transpose(xyz2, (0, 2, 1))
    k1, k2 = pl.pallas_call(
        _tc_body,
        grid=(B,),
        in_specs=[
            pl.BlockSpec((None, _N, 3), lambda b: (b, 0, 0)),
            pl.BlockSpec((None, 3, _N), lambda b: (b, 0, 0)),
        ],
        out_specs=[
            pl.BlockSpec((None, _N, 1), lambda b: (b, 0, 0)),
            pl.BlockSpec((None, 1, _N), lambda b: (b, 0, 0)),
        ],
        out_shape=[
            jax.ShapeDtypeStruct((B, _N, 1), jnp.float32),
            jax.ShapeDtypeStruct((B, 1, _N), jnp.float32),
        ],
    )(xyz1, x2t)
    parts = _sc_density(k1.reshape(B, _N), k2.reshape(B, _N))
    return 1.0 - jnp.sum(parts) / (2.0 * _N * B)
